# baseline (speedup 1.0000x reference)
import functools

import jax
import jax.numpy as jnp
from jax import lax
from jax.experimental import pallas as pl
from jax.experimental.pallas import tpu as pltpu
from jax.experimental.pallas import tpu_sc as plsc

BATCH = 4096
NBHD = 200
DIM = 64
LANES = 16
NVEC = DIM // LANES

GL = 100
GPB = NBHD // GL
CB = 4
GPC = CB * GPB


@functools.cache
def _build_sc_kernel():
    info = plsc.get_sparse_core_info()
    nw = info.num_cores * info.num_subcores
    rows_per_tile = BATCH // nw
    chunks = rows_per_tile // CB
    g_per_tile = rows_per_tile * GPB

    mesh = plsc.VectorSubcoreMesh(core_axis_name="c", subcore_axis_name="s")

    @functools.partial(
        pl.kernel,
        out_type=jax.ShapeDtypeStruct((BATCH, DIM), jnp.float32),
        scratch_types=[
            pltpu.VMEM((GPC, GL), jnp.int32),
            pltpu.VMEM((GPC, GL, DIM), jnp.float32),
            pltpu.VMEM((CB, DIM), jnp.float32),
            pltpu.SemaphoreType.DMA,
        ],
        mesh=mesh,
        compiler_params=pltpu.CompilerParams(use_tc_tiling_on_sc=False),
    )
    def k(idx_hbm, table_hbm, out_hbm, idx_v, rows_v, out_v, sem):
        wid = lax.axis_index("s") * info.num_cores + lax.axis_index("c")
        gbase = wid * g_per_tile
        rbase = wid * rows_per_tile

        def chunk_body(c, carry):
            pltpu.sync_copy(idx_hbm.at[pl.ds(gbase + c * GPC, GPC)], idx_v)
            cps = [
                pltpu.async_copy(table_hbm.at[idx_v.at[g]], rows_v.at[g], sem)
                for g in range(GPC)
            ]
            for cp in cps:
                cp.wait()
            for r in range(CB):
                accs = tuple(jnp.zeros((LANES,), jnp.float32) for _ in range(NVEC))
                for h in range(GPB):
                    g = r * GPB + h

                    def body(n, a, g=g):
                        return tuple(
                            a[d] + rows_v[g, n, pl.ds(LANES * d, LANES)]
                            for d in range(NVEC)
                        )

                    accs = lax.fori_loop(0, GL, body, accs)
                for d in range(NVEC):
                    out_v[r, pl.ds(LANES * d, LANES)] = accs[d] / float(NBHD)
            pltpu.sync_copy(out_v, out_hbm.at[pl.ds(rbase + c * CB, CB)])
            return carry

        lax.fori_loop(0, chunks, chunk_body, 0)

    return k


def kernel(relation_indices, relation_table):
    idx = relation_indices[..., 0].astype(jnp.int32)
    idx = idx.reshape(BATCH * NBHD // GL, GL)
    return _build_sc_kernel()(idx, relation_table)


# speedup vs baseline: 1.0033x; 1.0033x over previous
import functools

import jax
import jax.numpy as jnp
from jax import lax
from jax.experimental import pallas as pl
from jax.experimental.pallas import tpu as pltpu
from jax.experimental.pallas import tpu_sc as plsc

BATCH = 4096
NBHD = 200
DIM = 64
LANES = 16
NVEC = DIM // LANES

CB = 4


@functools.cache
def _build_sc_kernel():
    info = plsc.get_sparse_core_info()
    nw = info.num_cores * info.num_subcores
    rows_per_tile = BATCH // nw
    chunks = rows_per_tile // CB

    mesh = plsc.VectorSubcoreMesh(core_axis_name="c", subcore_axis_name="s")

    @functools.partial(
        pl.kernel,
        out_type=jax.ShapeDtypeStruct((BATCH, DIM), jnp.float32),
        scratch_types=[
            pltpu.VMEM((CB, NBHD), jnp.int32),
            pltpu.VMEM((CB, NBHD, DIM), jnp.float32),
            pltpu.VMEM((CB, DIM), jnp.float32),
            pltpu.SemaphoreType.DMA,
        ],
        mesh=mesh,
        compiler_params=pltpu.CompilerParams(use_tc_tiling_on_sc=False),
    )
    def k(idx_hbm, table_hbm, out_hbm, idx_v, rows_v, out_v, sem):
        wid = lax.axis_index("s") * info.num_cores + lax.axis_index("c")
        rbase = wid * rows_per_tile

        def chunk_body(c, carry):
            base = rbase + c * CB
            pltpu.sync_copy(idx_hbm.at[pl.ds(base, CB)], idx_v)
            cps = [
                pltpu.async_copy(table_hbm.at[idx_v.at[r]], rows_v.at[r], sem)
                for r in range(CB)
            ]
            for cp in cps:
                cp.wait()
            for r in range(CB):
                accs = tuple(jnp.zeros((LANES,), jnp.float32) for _ in range(NVEC))

                def body(n, a, r=r):
                    return tuple(
                        a[d] + rows_v[r, n, pl.ds(LANES * d, LANES)]
                        for d in range(NVEC)
                    )

                accs = lax.fori_loop(0, NBHD, body, accs)
                for d in range(NVEC):
                    out_v[r, pl.ds(LANES * d, LANES)] = accs[d] / float(NBHD)
            pltpu.sync_copy(out_v, out_hbm.at[pl.ds(base, CB)])
            return carry

        lax.fori_loop(0, chunks, chunk_body, 0)

    return k


def kernel(relation_indices, relation_table):
    idx = relation_indices[..., 0].astype(jnp.int32)
    return _build_sc_kernel()(idx, relation_table)
